# baseline (speedup 1.0000x reference)
import jax
import jax.numpy as jnp
from jax.experimental import pallas as pl
from jax.experimental.pallas import tpu as pltpu

_N, _C = 100000, 80
_ROWS = 2000
_GRID = _N // _ROWS

_L1P_COEF = (9.0837868449e-08, 9.9999145457e-01, -4.9980116320e-01,
             3.3133400573e-01, -2.3919071732e-01, 1.6478349730e-01,
             -9.2313768670e-02, 3.4418593521e-02, -6.0748776437e-03)
_RCP_COEF = (9.9999989379e-01, -9.9998777872e-01, 9.9965117021e-01,
             -9.9566916706e-01, 9.7079622569e-01, -8.7974872665e-01,
             6.7449814969e-01, -3.8608484079e-01, 1.4005623342e-01,
             -2.3511233453e-02)


def _polyval(coef, t):
    acc = jnp.full_like(t, coef[-1])
    for c in coef[-2::-1]:
        acc = acc * t + c
    return acc


def _qfl_body(pred_ref, lab_ref, sc_ref, out_ref):
    i = pl.program_id(0)
    x = pred_ref[...]
    lab = lab_ref[0, 0, :]
    s = sc_ref[0, 0, :]

    ax = jnp.abs(x)
    t = jnp.exp(-ax)
    l1p = _polyval(_L1P_COEF, t)
    rc = _polyval(_RCP_COEF, t)
    relu = jnp.maximum(x, 0.0)
    sig = jnp.where(x >= 0, rc, t * rc)

    neg = (relu + l1p) * sig * sig
    sb = s[:, None]
    d = sb - sig
    pos = (relu - x * sb + l1p) * d * d

    col = jax.lax.broadcasted_iota(jnp.int32, x.shape, 1)
    m = col == lab[:, None]
    part = jnp.sum(jnp.where(m, pos, neg))

    @pl.when(i == 0)
    def _init():
        out_ref[0, 0] = part

    @pl.when(i > 0)
    def _acc():
        out_ref[0, 0] += part


def kernel(pred, label, score):
    lab3 = label.astype(jnp.int32).reshape(_GRID, 1, _ROWS)
    sc3 = score.reshape(_GRID, 1, _ROWS)
    total = pl.pallas_call(
        _qfl_body,
        grid=(_GRID,),
        in_specs=[
            pl.BlockSpec((_ROWS, _C), lambda i: (i, 0)),
            pl.BlockSpec((1, 1, _ROWS), lambda i: (i, 0, 0)),
            pl.BlockSpec((1, 1, _ROWS), lambda i: (i, 0, 0)),
        ],
        out_specs=pl.BlockSpec(memory_space=pltpu.SMEM),
        out_shape=jax.ShapeDtypeStruct((1, 1), jnp.float32),
    )(pred, lab3, sc3)
    return total[0, 0] / _N


# speedup vs baseline: 1.4866x; 1.4866x over previous
import jax
import jax.numpy as jnp
from jax.experimental import pallas as pl
from jax.experimental.pallas import tpu as pltpu

_N, _C = 100000, 80
_ROWS = 2000
_GRID = _N // _ROWS

_L1P_COEF = (9.0837868449e-08, 9.9999145457e-01, -4.9980116320e-01,
             3.3133400573e-01, -2.3919071732e-01, 1.6478349730e-01,
             -9.2313768670e-02, 3.4418593521e-02, -6.0748776437e-03)
_RCP_COEF = (9.9999989379e-01, -9.9998777872e-01, 9.9965117021e-01,
             -9.9566916706e-01, 9.7079622569e-01, -8.7974872665e-01,
             6.7449814969e-01, -3.8608484079e-01, 1.4005623342e-01,
             -2.3511233453e-02)


def _polyval(coef, t):
    acc = jnp.full_like(t, coef[-1])
    for c in coef[-2::-1]:
        acc = acc * t + c
    return acc


def _qfl_body(pred_ref, lab_ref, sc_ref, out_ref):
    i = pl.program_id(0)
    x = pred_ref[...]
    lab = lab_ref[0, 0, :]
    s = sc_ref[0, 0, :]

    sig = 0.5 * jnp.tanh(0.5 * x) + 0.5
    sp = jnp.where(x > 12.0, x, -jnp.log(1.0 - sig))

    neg = sp * sig * sig
    sb = s[:, None]
    d = sb - sig
    pos = (sp - x * sb) * d * d

    col = jax.lax.broadcasted_iota(jnp.int32, x.shape, 1)
    m = col == lab[:, None]
    part = jnp.sum(jnp.where(m, pos, neg))

    @pl.when(i == 0)
    def _init():
        out_ref[0, 0] = part

    @pl.when(i > 0)
    def _acc():
        out_ref[0, 0] += part


def kernel(pred, label, score):
    lab3 = label.astype(jnp.int32).reshape(_GRID, 1, _ROWS)
    sc3 = score.reshape(_GRID, 1, _ROWS)
    total = pl.pallas_call(
        _qfl_body,
        grid=(_GRID,),
        in_specs=[
            pl.BlockSpec((_ROWS, _C), lambda i: (i, 0)),
            pl.BlockSpec((1, 1, _ROWS), lambda i: (i, 0, 0)),
            pl.BlockSpec((1, 1, _ROWS), lambda i: (i, 0, 0)),
        ],
        out_specs=pl.BlockSpec(memory_space=pltpu.SMEM),
        out_shape=jax.ShapeDtypeStruct((1, 1), jnp.float32),
    )(pred, lab3, sc3)
    return total[0, 0] / _N


# speedup vs baseline: 1.6737x; 1.1258x over previous
import jax
import jax.numpy as jnp
from jax.experimental import pallas as pl
from jax.experimental.pallas import tpu as pltpu

_N, _C = 100000, 80
_ROWS = 10000
_GRID = _N // _ROWS

_L1P_COEF = (9.0837868449e-08, 9.9999145457e-01, -4.9980116320e-01,
             3.3133400573e-01, -2.3919071732e-01, 1.6478349730e-01,
             -9.2313768670e-02, 3.4418593521e-02, -6.0748776437e-03)
_RCP_COEF = (9.9999989379e-01, -9.9998777872e-01, 9.9965117021e-01,
             -9.9566916706e-01, 9.7079622569e-01, -8.7974872665e-01,
             6.7449814969e-01, -3.8608484079e-01, 1.4005623342e-01,
             -2.3511233453e-02)


def _polyval(coef, t):
    acc = jnp.full_like(t, coef[-1])
    for c in coef[-2::-1]:
        acc = acc * t + c
    return acc


def _qfl_body(pred_ref, lab_ref, sc_ref, out_ref):
    i = pl.program_id(0)
    x = pred_ref[...]
    lab = lab_ref[0, 0, :]
    s = sc_ref[0, 0, :]

    sig = 0.5 * jnp.tanh(0.5 * x) + 0.5
    sp = jnp.where(x > 12.0, x, -jnp.log(1.0 - sig))

    neg = sp * sig * sig
    sb = s[:, None]
    d = sb - sig
    pos = (sp - x * sb) * d * d

    col = jax.lax.broadcasted_iota(jnp.int32, x.shape, 1)
    m = col == lab[:, None]
    part = jnp.sum(jnp.where(m, pos, neg))

    @pl.when(i == 0)
    def _init():
        out_ref[0, 0] = part

    @pl.when(i > 0)
    def _acc():
        out_ref[0, 0] += part


def kernel(pred, label, score):
    lab3 = label.astype(jnp.int32).reshape(_GRID, 1, _ROWS)
    sc3 = score.reshape(_GRID, 1, _ROWS)
    total = pl.pallas_call(
        _qfl_body,
        grid=(_GRID,),
        in_specs=[
            pl.BlockSpec((_ROWS, _C), lambda i: (i, 0)),
            pl.BlockSpec((1, 1, _ROWS), lambda i: (i, 0, 0)),
            pl.BlockSpec((1, 1, _ROWS), lambda i: (i, 0, 0)),
        ],
        out_specs=pl.BlockSpec(memory_space=pltpu.SMEM),
        out_shape=jax.ShapeDtypeStruct((1, 1), jnp.float32),
    )(pred, lab3, sc3)
    return total[0, 0] / _N
